# baseline (speedup 1.0000x reference)
import functools

import jax
import jax.numpy as jnp
from jax import lax
from jax.experimental import pallas as pl
from jax.experimental.pallas import tpu as pltpu
from jax.experimental.pallas import tpu_sc as plsc

B = 128
S = 199
D = 128
N = S * B

_NW = 32
_CHUNK = 112
_CPW = 8
_ROWS_W = _CHUNK * _CPW
_NP = _NW * _ROWS_W
_IDXR = _NP // _CHUNK

_F32 = jnp.float32


_BC = 8


def _lastocc_body(q_ref, c_ref, combo_ref, gapp_ref, gaps_ref):
    q = q_ref[:]
    c = c_ref[:]
    tpr = lax.broadcasted_iota(jnp.int32, (_BC, S, S), 2)
    tcu = lax.broadcasted_iota(jnp.int32, (_BC, S, S), 1)
    lt = tpr < tcu
    eqq = (q[:, :, None] == q[:, None, :]) & lt
    lbp = jnp.max(jnp.where(eqq, tpr, 0), axis=2)
    eqc = (c[:, :, None] == c[:, None, :]) & lt
    lbs = jnp.max(jnp.where(eqc, tpr, 0), axis=2)
    tv = lax.broadcasted_iota(jnp.int32, (_BC, S), 1)
    combo_ref[:] = lbp * 256 + lbs
    gapp_ref[:] = tv - lbp
    gaps_ref[:] = tv - lbs


def _lastocc(q, c):
    grid = (B // _BC,)
    spec = pl.BlockSpec((_BC, S), lambda i: (i, 0))
    return pl.pallas_call(
        _lastocc_body,
        grid=grid,
        in_specs=[spec, spec],
        out_specs=[spec, spec, spec],
        out_shape=[jax.ShapeDtypeStruct((B, S), jnp.int32)] * 3,
    )(q, c)


def _tables_body(te_ref, pfw2_ref, sfw2_ref, afw2_ref, pfb_ref, sfb_ref,
                 afb_ref, ans_ref, c3_ref, pft_ref, sft_ref, afc_ref,
                 ansc3_ref):
    te = te_ref[:]
    pft_ref[:] = jnp.dot(te, pfw2_ref[:], preferred_element_type=_F32) + pfb_ref[:]
    sft_ref[:] = jnp.dot(te, sfw2_ref[:], preferred_element_type=_F32) + sfb_ref[:]
    afc_ref[:] = jnp.dot(te[1:2], afw2_ref[:], preferred_element_type=_F32) + afb_ref[:]
    ansc3_ref[:] = jnp.dot(ans_ref[:], c3_ref[:], preferred_element_type=_F32)


def _tables(time_embed, pfw2, sfw2, afw2, pfb, sfb, afb, ans_embed, c3):
    return pl.pallas_call(
        _tables_body,
        out_shape=[
            jax.ShapeDtypeStruct((200, D), _F32),
            jax.ShapeDtypeStruct((200, D), _F32),
            jax.ShapeDtypeStruct((1, D), _F32),
            jax.ShapeDtypeStruct((2, 3 * D), _F32),
        ],
    )(time_embed, pfw2, sfw2, afw2, pfb, sfb, afb, ans_embed, c3)


def _sc_gather(qf, qhif, cf, pro_embed, skill_embed, akt_change,
               akt_diff2d):
    mesh = plsc.VectorSubcoreMesh(core_axis_name="c", subcore_axis_name="s")

    @functools.partial(
        pl.kernel,
        mesh=mesh,
        out_type=[
            jax.ShapeDtypeStruct((_NP, D), _F32),
            jax.ShapeDtypeStruct((_NP, D), _F32),
            jax.ShapeDtypeStruct((_NP, D), _F32),
            jax.ShapeDtypeStruct((_NP, D), _F32),
        ],
        scratch_types=[
            pltpu.VMEM((_CPW, _CHUNK), jnp.int32),
            pltpu.VMEM((_CPW, _CHUNK), jnp.int32),
            pltpu.VMEM((_CPW, _CHUNK), jnp.int32),
            pltpu.VMEM((2, _CHUNK, D), _F32),
            pltpu.VMEM((2, _CHUNK, D), _F32),
            pltpu.VMEM((2, _CHUNK, D), _F32),
            pltpu.VMEM((2, _CHUNK, D), _F32),
            pltpu.SemaphoreType.DMA,
            pltpu.SemaphoreType.DMA,
            pltpu.SemaphoreType.DMA,
        ],
    )
    def k(qh, qhih, ch, proh, sklh, chgh, difh,
          opro, oskl, ochg, odif,
          qi, qhii, ci, bpro, bskl, bchg, bdif, gsem0, gsem1, wsem):
        wid = lax.axis_index("s") * 2 + lax.axis_index("c")
        pltpu.sync_copy(qh.at[pl.ds(wid * _CPW, _CPW)], qi)
        pltpu.sync_copy(qhih.at[pl.ds(wid * _CPW, _CPW)], qhii)
        pltpu.sync_copy(ch.at[pl.ds(wid * _CPW, _CPW)], ci)
        gsems = (gsem0, gsem1)

        def fire(c, s):
            gs = gsems[s]
            return (pltpu.async_copy(proh.at[qi.at[c]], bpro.at[s], gs),
                    pltpu.async_copy(sklh.at[ci.at[c]], bskl.at[s], gs),
                    pltpu.async_copy(chgh.at[ci.at[c]], bchg.at[s], gs),
                    pltpu.async_copy(difh.at[qhii.at[c]], bdif.at[s], gs))

        pend = fire(0, 0)
        wrs = {0: [], 1: []}
        for c in range(_CPW):
            s = c % 2
            for d in pend:
                d.wait()
            if c + 1 < _CPW:
                for w in wrs[1 - s]:
                    w.wait()
                wrs[1 - s] = []
                pend = fire(c + 1, 1 - s)
            dst = pl.ds(wid * _ROWS_W + c * _CHUNK, _CHUNK)
            wrs[s] = [
                pltpu.async_copy(bpro.at[s], opro.at[dst], wsem),
                pltpu.async_copy(bskl.at[s], oskl.at[dst], wsem),
                pltpu.async_copy(bchg.at[s], ochg.at[dst], wsem),
                pltpu.async_copy(bdif.at[s], odif.at[dst], wsem),
            ]
        for w in wrs[0] + wrs[1]:
            w.wait()

    return k(qf, qhif, cf, pro_embed, skill_embed, akt_change, akt_diff2d)


_TB = 8


def _xform_body(pro_ref, skl_ref, chg_ref, difr_ref, qlane_ref, r_ref,
                c3_ref, w1d_ref, b3_ref, b1_ref, ansc3_ref, xps_ref,
                outx_ref):
    lane = lax.broadcasted_iota(jnp.int32, (_TB, B, D), 2)
    dmask = lane == qlane_ref[:][:, :, None]
    dif = jnp.sum(jnp.where(dmask, difr_ref[:].reshape(_TB, B, D), 0.0),
                  axis=2, keepdims=True)
    npe3 = (pro_ref[:].reshape(_TB, B, D) + skl_ref[:].reshape(_TB, B, D)
            + dif * chg_ref[:].reshape(_TB, B, D))
    npe2 = npe3.reshape(_TB * B, D)
    outx2 = jnp.dot(npe2, w1d_ref[:], preferred_element_type=_F32) + b1_ref[:]
    ansc = ansc3_ref[:]
    r = r_ref[:]
    sel = jnp.where(r[:, :, None] > 0, ansc[1:2][None], ansc[0:1][None])
    xyz2 = (jnp.dot(npe2, c3_ref[:], preferred_element_type=_F32)
            + b3_ref[:] + sel.reshape(_TB * B, 3 * D))
    xps_ref[:] = xyz2.reshape(_TB, B, 3 * D)
    outx_ref[:] = outx2.reshape(_TB, B, D)


def _xform(pro_r, skl_r, chg_r, difr_r, qlane, rt, c3, w1d, b3, b1, ansc3):
    grid = (pl.cdiv(S, _TB),)
    row3 = pl.BlockSpec((_TB, B, D), lambda i: (i, 0, 0))
    rowf = pl.BlockSpec((_TB * B, D), lambda i: (i, 0))
    row2 = pl.BlockSpec((_TB, B), lambda i: (i, 0))
    full = lambda shp: pl.BlockSpec(shp, lambda i: tuple(0 for _ in shp))
    return pl.pallas_call(
        _xform_body,
        grid=grid,
        in_specs=[rowf, rowf, rowf, rowf, row2, row2,
                  full((D, 3 * D)), full((D, D)), full((1, 3 * D)),
                  full((1, D)), full((2, 3 * D))],
        out_specs=[pl.BlockSpec((_TB, B, 3 * D), lambda i: (i, 0, 0)), row3],
        out_shape=[jax.ShapeDtypeStruct((S, B, 3 * D), _F32),
                   jax.ShapeDtypeStruct((S, B, D), _F32)],
    )(pro_r, skl_r, chg_r, difr_r, qlane, rt, c3, w1d, b3, b1, ansc3)


def _gaprows_body(gapp_ref, gaps_ref, pft_ref, sft_ref, pfg_ref, sfg_ref):
    giota = lax.broadcasted_iota(jnp.int32, (_TB, B, 200), 2)
    ohp = (gapp_ref[:][:, :, None] == giota).astype(_F32).reshape(
        _TB * B, 200)
    ohs = (gaps_ref[:][:, :, None] == giota).astype(_F32).reshape(
        _TB * B, 200)
    pfg_ref[:] = jnp.dot(ohp, pft_ref[:],
                         preferred_element_type=_F32).reshape(_TB, B, D)
    sfg_ref[:] = jnp.dot(ohs, sft_ref[:],
                         preferred_element_type=_F32).reshape(_TB, B, D)


def _gaprows(gappt, gapst, pft, sft):
    grid = (pl.cdiv(S, _TB),)
    row3 = pl.BlockSpec((_TB, B, D), lambda i: (i, 0, 0))
    row2 = pl.BlockSpec((_TB, B), lambda i: (i, 0))
    full = lambda shp: pl.BlockSpec(shp, lambda i: tuple(0 for _ in shp))
    return pl.pallas_call(
        _gaprows_body,
        grid=grid,
        in_specs=[row2, row2, full((200, D)), full((200, D))],
        out_specs=[row3, row3],
        out_shape=[jax.ShapeDtypeStruct((S, B, D), _F32),
                   jax.ShapeDtypeStruct((S, B, D), _F32)],
    )(gappt, gapst, pft, sft)


def _seq_body(combo_ref, xps_ref, pfg_ref, sfg_ref, lbpn_ref, lbsn_ref,
              pf1_ref, sf1_ref, af1_ref, ps1_ref, ss1_ref, as1_ref,
              afc_ref, p0_ref, s0_ref, a0_ref,
              olbas_ref, olbps_ref, olbss_ref,
              histp, hists, allst, gbp, gbs):
    t = pl.program_id(0)

    @pl.when(t == 0)
    def _init():
        gbp[:] = jnp.broadcast_to(p0_ref[:], (B, D))
        gbs[:] = jnp.broadcast_to(s0_ref[:], (B, D))
        allst[:] = jnp.broadcast_to(a0_ref[:], (B, D))

    gp = gbp[:]
    gs = gbs[:]

    tt = jnp.minimum(t + 1, S - 1)

    def bbody(b, carry):
        v = combo_ref[b, tt]
        gbp[b, :] = histp[v >> 8, b, :]
        gbs[b, :] = hists[v & 255, b, :]
        return carry

    lax.fori_loop(0, B, bbody, 0, unroll=True)

    pf = jax.nn.sigmoid(
        jnp.dot(gp, pf1_ref[:], preferred_element_type=_F32) + pfg_ref[0])
    lbps = gp * pf
    sf = jax.nn.sigmoid(
        jnp.dot(gs, sf1_ref[:], preferred_element_type=_F32) + sfg_ref[0])
    lbss = gs * sf
    a = allst[:]
    af = jax.nn.sigmoid(
        jnp.dot(a, af1_ref[:], preferred_element_type=_F32) + afc_ref[:])
    lbas = a * af
    olbas_ref[0] = lbas
    olbps_ref[0] = lbps
    olbss_ref[0] = lbss
    x = xps_ref[0]
    allst[:] = lbas + jnp.tanh(
        jnp.dot(lbas, as1_ref[:], preferred_element_type=_F32) + x[:, 0:D])
    newp = lbps + jnp.tanh(
        jnp.dot(lbps, ps1_ref[:], preferred_element_type=_F32) + x[:, D:2 * D])
    news = lbss + jnp.tanh(
        jnp.dot(lbss, ss1_ref[:], preferred_element_type=_F32) + x[:, 2 * D:])
    histp[t] = newp
    hists[t] = news
    gbp[:] = jnp.where(lbpn_ref[0] == t, newp, gbp[:])
    gbs[:] = jnp.where(lbsn_ref[0] == t, news, gbs[:])


def _seq(combo, xps, pfg3, sfg3, lbpn, lbsn, pf1, sf1, af1, ps1, ss1, as1,
         afc, p0, s0, a0):
    row3 = pl.BlockSpec((1, B, D), lambda t, _c: (t, 0, 0))
    nxt = pl.BlockSpec((1, B, 1),
                       lambda t, _c: (jnp.minimum(t + 1, S - 1), 0, 0))
    full = lambda shp: pl.BlockSpec(shp, lambda t, _c: tuple(0 for _ in shp))
    grid_spec = pltpu.PrefetchScalarGridSpec(
        num_scalar_prefetch=1,
        grid=(S,),
        in_specs=[pl.BlockSpec((1, B, 3 * D), lambda t, _c: (t, 0, 0)),
                  row3, row3, nxt, nxt,
                  full((D, D)), full((D, D)), full((D, D)), full((D, D)),
                  full((D, D)), full((D, D)), full((1, D)), full((1, D)),
                  full((1, D)), full((1, D))],
        out_specs=[row3, row3, row3],
        scratch_shapes=[
            pltpu.VMEM((S, B, D), _F32),
            pltpu.VMEM((S, B, D), _F32),
            pltpu.VMEM((B, D), _F32),
            pltpu.VMEM((B, D), _F32),
            pltpu.VMEM((B, D), _F32),
        ],
    )
    return pl.pallas_call(
        _seq_body,
        grid_spec=grid_spec,
        out_shape=[jax.ShapeDtypeStruct((S, B, D), _F32)] * 3,
    )(combo, xps, pfg3, sfg3, lbpn, lbsn, pf1, sf1, af1, ps1, ss1, as1, afc,
      p0, s0, a0)


def _head_body(lbas_ref, lbps_ref, lbss_ref, outx_ref, w1_ref, w2_ref,
               b2_ref, p_ref):
    m = jnp.concatenate([lbas_ref[:], lbps_ref[:], lbss_ref[:]],
                        axis=2).reshape(_TB * B, 3 * D)
    h = jnp.maximum(
        jnp.dot(m, w1_ref[:], preferred_element_type=_F32)
        + outx_ref[:].reshape(_TB * B, D), 0.0)
    p = jax.nn.sigmoid(
        jnp.dot(h, w2_ref[:], preferred_element_type=_F32) + b2_ref[:])
    p_ref[:] = p.reshape(_TB, B)


def _head(lbas, lbps, lbss, outx, w1abc, w2, b2):
    grid = (pl.cdiv(S, _TB),)
    row3 = pl.BlockSpec((_TB, B, D), lambda i: (i, 0, 0))
    full = lambda shp: pl.BlockSpec(shp, lambda i: tuple(0 for _ in shp))
    return pl.pallas_call(
        _head_body,
        grid=grid,
        in_specs=[row3, row3, row3, row3,
                  full((3 * D, D)), full((D, 1)), full((1, 1))],
        out_specs=pl.BlockSpec((_TB, B), lambda i: (i, 0)),
        out_shape=jax.ShapeDtypeStruct((S, B), _F32),
    )(lbas, lbps, lbss, outx, w1abc, w2, b2)


def _flatpad(x_bs):
    f = x_bs.T.reshape(N)
    f = jnp.concatenate([f, jnp.zeros((_NP - N,), jnp.int32)])
    return f.reshape(_IDXR, _CHUNK)


def kernel(qseqs, cseqs, rseqs, shft_qseqs, shft_cseqs, shft_rseqs,
           pro_embed, skill_embed, ans_embed, time_embed, ls_state,
           pro_state_init, skill_state_init, akt_pro_diff, akt_pro_change,
           out_w1, out_b1, out_w2, out_b2, pf_w, pf_b, ps_w, ps_b, af_w,
           af_b, sf_w, sf_b, ss_w, ss_b, as_w, as_b):
    pf1 = pf_w[:, :D].T
    sf1 = sf_w[:, :D].T
    af1 = af_w[:, :D].T
    ps1 = ps_w[:, :D].T
    ss1 = ss_w[:, :D].T
    as1 = as_w[:, :D].T
    pfw2 = pf_w[:, D:].T
    sfw2 = sf_w[:, D:].T
    afw2 = af_w[:, D:].T
    c3 = jnp.concatenate([as_w[:, D:].T, ps_w[:, D:].T, ss_w[:, D:].T], 1)
    b3 = jnp.concatenate([as_b, ps_b, ss_b]).reshape(1, 3 * D)
    w1abc = out_w1[:, :3 * D].T
    w1d = out_w1[:, 3 * D:].T
    b1 = out_b1.reshape(1, D)
    w2 = out_w2.T
    b2 = out_b2.reshape(1, 1)

    combo, gapp, gaps = _lastocc(shft_qseqs, shft_cseqs)

    pft, sft, afc, ansc3 = _tables(time_embed, pfw2, sfw2, afw2,
                                   pf_b.reshape(1, D), sf_b.reshape(1, D),
                                   af_b.reshape(1, D), ans_embed, c3)

    qf = _flatpad(shft_qseqs)
    qhif = _flatpad(shft_qseqs >> 7)
    cf = _flatpad(shft_cseqs)
    dif2d = jnp.concatenate(
        [akt_pro_diff.reshape(-1),
         jnp.zeros((782 * 128 - akt_pro_diff.shape[0],), _F32)]).reshape(
             782, 128)
    pro_r, skl_r, chg_r, difr_r = _sc_gather(
        qf, qhif, cf, pro_embed, skill_embed, akt_pro_change, dif2d)

    pfg3, sfg3 = _gaprows(gapp.T, gaps.T, pft, sft)

    rt = shft_rseqs.T
    qlane = (shft_qseqs & 127).T
    xps, outx = _xform(pro_r, skl_r, chg_r, difr_r, qlane, rt, c3, w1d, b3,
                       b1, ansc3)

    lbpn = (combo >> 8).T[:, :, None]
    lbsn = (combo & 255).T[:, :, None]
    lbas, lbps, lbss = _seq(combo, xps, pfg3, sfg3, lbpn, lbsn, pf1, sf1,
                            af1, ps1, ss1, as1, afc, pro_state_init[0:1],
                            skill_state_init[0:1], ls_state)

    pt = _head(lbas, lbps, lbss, outx, w1abc, w2, b2)
    return pt.T


# speedup vs baseline: 1.0223x; 1.0223x over previous
import functools

import jax
import jax.numpy as jnp
from jax import lax
from jax.experimental import pallas as pl
from jax.experimental.pallas import tpu as pltpu
from jax.experimental.pallas import tpu_sc as plsc

B = 128
S = 199
D = 128
N = S * B

_NW = 32
_CHUNK = 112
_CPW = 8
_ROWS_W = _CHUNK * _CPW
_NP = _NW * _ROWS_W
_IDXR = _NP // _CHUNK

_F32 = jnp.float32


_BC = 8


def _lastocc_body(q_ref, c_ref, combo_ref, gapp_ref, gaps_ref):
    q = q_ref[:]
    c = c_ref[:]
    tpr = lax.broadcasted_iota(jnp.int32, (_BC, S, S), 2)
    tcu = lax.broadcasted_iota(jnp.int32, (_BC, S, S), 1)
    lt = tpr < tcu
    eqq = (q[:, :, None] == q[:, None, :]) & lt
    lbp = jnp.max(jnp.where(eqq, tpr, 0), axis=2)
    eqc = (c[:, :, None] == c[:, None, :]) & lt
    lbs = jnp.max(jnp.where(eqc, tpr, 0), axis=2)
    tv = lax.broadcasted_iota(jnp.int32, (_BC, S), 1)
    combo_ref[:] = lbp * 256 + lbs
    gapp_ref[:] = tv - lbp
    gaps_ref[:] = tv - lbs


def _lastocc(q, c):
    grid = (B // _BC,)
    spec = pl.BlockSpec((_BC, S), lambda i: (i, 0))
    return pl.pallas_call(
        _lastocc_body,
        grid=grid,
        in_specs=[spec, spec],
        out_specs=[spec, spec, spec],
        out_shape=[jax.ShapeDtypeStruct((B, S), jnp.int32)] * 3,
    )(q, c)


def _tables_body(te_ref, pfw2_ref, sfw2_ref, afw2_ref, pfb_ref, sfb_ref,
                 afb_ref, ans_ref, c3_ref, pft_ref, sft_ref, afc_ref,
                 ansc3_ref):
    te = te_ref[:]
    pft_ref[:] = jnp.dot(te, pfw2_ref[:], preferred_element_type=_F32) + pfb_ref[:]
    sft_ref[:] = jnp.dot(te, sfw2_ref[:], preferred_element_type=_F32) + sfb_ref[:]
    afc_ref[:] = jnp.dot(te[1:2], afw2_ref[:], preferred_element_type=_F32) + afb_ref[:]
    ansc3_ref[:] = jnp.dot(ans_ref[:], c3_ref[:], preferred_element_type=_F32)


def _tables(time_embed, pfw2, sfw2, afw2, pfb, sfb, afb, ans_embed, c3):
    return pl.pallas_call(
        _tables_body,
        out_shape=[
            jax.ShapeDtypeStruct((200, D), _F32),
            jax.ShapeDtypeStruct((200, D), _F32),
            jax.ShapeDtypeStruct((1, D), _F32),
            jax.ShapeDtypeStruct((2, 3 * D), _F32),
        ],
    )(time_embed, pfw2, sfw2, afw2, pfb, sfb, afb, ans_embed, c3)


def _sc_gather(qf, qhif, cf, pro_embed, skill_embed, akt_change,
               akt_diff2d):
    mesh = plsc.VectorSubcoreMesh(core_axis_name="c", subcore_axis_name="s")

    @functools.partial(
        pl.kernel,
        mesh=mesh,
        out_type=[
            jax.ShapeDtypeStruct((_NP, D), _F32),
            jax.ShapeDtypeStruct((_NP, D), _F32),
            jax.ShapeDtypeStruct((_NP, D), _F32),
            jax.ShapeDtypeStruct((_NP, D), _F32),
        ],
        scratch_types=[
            pltpu.VMEM((_CPW, _CHUNK), jnp.int32),
            pltpu.VMEM((_CPW, _CHUNK), jnp.int32),
            pltpu.VMEM((_CPW, _CHUNK), jnp.int32),
            pltpu.VMEM((2, _CHUNK, D), _F32),
            pltpu.VMEM((2, _CHUNK, D), _F32),
            pltpu.VMEM((2, _CHUNK, D), _F32),
            pltpu.VMEM((2, _CHUNK, D), _F32),
            pltpu.SemaphoreType.DMA,
            pltpu.SemaphoreType.DMA,
            pltpu.SemaphoreType.DMA,
        ],
    )
    def k(qh, qhih, ch, proh, sklh, chgh, difh,
          opro, oskl, ochg, odif,
          qi, qhii, ci, bpro, bskl, bchg, bdif, gsem0, gsem1, wsem):
        wid = lax.axis_index("s") * 2 + lax.axis_index("c")
        pltpu.sync_copy(qh.at[pl.ds(wid * _CPW, _CPW)], qi)
        pltpu.sync_copy(qhih.at[pl.ds(wid * _CPW, _CPW)], qhii)
        pltpu.sync_copy(ch.at[pl.ds(wid * _CPW, _CPW)], ci)
        gsems = (gsem0, gsem1)

        def fire(c, s):
            gs = gsems[s]
            return (pltpu.async_copy(proh.at[qi.at[c]], bpro.at[s], gs),
                    pltpu.async_copy(sklh.at[ci.at[c]], bskl.at[s], gs),
                    pltpu.async_copy(chgh.at[ci.at[c]], bchg.at[s], gs),
                    pltpu.async_copy(difh.at[qhii.at[c]], bdif.at[s], gs))

        pend = fire(0, 0)
        wrs = {0: [], 1: []}
        for c in range(_CPW):
            s = c % 2
            for d in pend:
                d.wait()
            if c + 1 < _CPW:
                for w in wrs[1 - s]:
                    w.wait()
                wrs[1 - s] = []
                pend = fire(c + 1, 1 - s)
            dst = pl.ds(wid * _ROWS_W + c * _CHUNK, _CHUNK)
            wrs[s] = [
                pltpu.async_copy(bpro.at[s], opro.at[dst], wsem),
                pltpu.async_copy(bskl.at[s], oskl.at[dst], wsem),
                pltpu.async_copy(bchg.at[s], ochg.at[dst], wsem),
                pltpu.async_copy(bdif.at[s], odif.at[dst], wsem),
            ]
        for w in wrs[0] + wrs[1]:
            w.wait()

    return k(qf, qhif, cf, pro_embed, skill_embed, akt_change, akt_diff2d)


_TB = 8


def _xform_body(pro_ref, skl_ref, chg_ref, difr_ref, qlane_ref, r_ref,
                c3_ref, w1d_ref, b3_ref, b1_ref, ansc3_ref, xps_ref,
                outx_ref):
    lane = lax.broadcasted_iota(jnp.int32, (_TB, B, D), 2)
    dmask = lane == qlane_ref[:][:, :, None]
    dif = jnp.sum(jnp.where(dmask, difr_ref[:].reshape(_TB, B, D), 0.0),
                  axis=2, keepdims=True)
    npe3 = (pro_ref[:].reshape(_TB, B, D) + skl_ref[:].reshape(_TB, B, D)
            + dif * chg_ref[:].reshape(_TB, B, D))
    npe2 = npe3.reshape(_TB * B, D)
    outx2 = jnp.dot(npe2, w1d_ref[:], preferred_element_type=_F32) + b1_ref[:]
    ansc = ansc3_ref[:]
    r = r_ref[:]
    sel = jnp.where(r[:, :, None] > 0, ansc[1:2][None], ansc[0:1][None])
    xyz2 = (jnp.dot(npe2, c3_ref[:], preferred_element_type=_F32)
            + b3_ref[:] + sel.reshape(_TB * B, 3 * D))
    xps_ref[:] = xyz2.reshape(_TB, B, 3 * D)
    outx_ref[:] = outx2.reshape(_TB, B, D)


def _xform(pro_r, skl_r, chg_r, difr_r, qlane, rt, c3, w1d, b3, b1, ansc3):
    grid = (pl.cdiv(S, _TB),)
    row3 = pl.BlockSpec((_TB, B, D), lambda i: (i, 0, 0))
    rowf = pl.BlockSpec((_TB * B, D), lambda i: (i, 0))
    row2 = pl.BlockSpec((_TB, B), lambda i: (i, 0))
    full = lambda shp: pl.BlockSpec(shp, lambda i: tuple(0 for _ in shp))
    return pl.pallas_call(
        _xform_body,
        grid=grid,
        in_specs=[rowf, rowf, rowf, rowf, row2, row2,
                  full((D, 3 * D)), full((D, D)), full((1, 3 * D)),
                  full((1, D)), full((2, 3 * D))],
        out_specs=[pl.BlockSpec((_TB, B, 3 * D), lambda i: (i, 0, 0)), row3],
        out_shape=[jax.ShapeDtypeStruct((S, B, 3 * D), _F32),
                   jax.ShapeDtypeStruct((S, B, D), _F32)],
    )(pro_r, skl_r, chg_r, difr_r, qlane, rt, c3, w1d, b3, b1, ansc3)


def _gaprows_body(gapp_ref, gaps_ref, pft_ref, sft_ref, pfg_ref, sfg_ref):
    giota = lax.broadcasted_iota(jnp.int32, (_TB, B, 200), 2)
    ohp = (gapp_ref[:][:, :, None] == giota).astype(_F32).reshape(
        _TB * B, 200)
    ohs = (gaps_ref[:][:, :, None] == giota).astype(_F32).reshape(
        _TB * B, 200)
    pfg_ref[:] = jnp.dot(ohp, pft_ref[:],
                         preferred_element_type=_F32).reshape(_TB, B, D)
    sfg_ref[:] = jnp.dot(ohs, sft_ref[:],
                         preferred_element_type=_F32).reshape(_TB, B, D)


def _gaprows(gappt, gapst, pft, sft):
    grid = (pl.cdiv(S, _TB),)
    row3 = pl.BlockSpec((_TB, B, D), lambda i: (i, 0, 0))
    row2 = pl.BlockSpec((_TB, B), lambda i: (i, 0))
    full = lambda shp: pl.BlockSpec(shp, lambda i: tuple(0 for _ in shp))
    return pl.pallas_call(
        _gaprows_body,
        grid=grid,
        in_specs=[row2, row2, full((200, D)), full((200, D))],
        out_specs=[row3, row3],
        out_shape=[jax.ShapeDtypeStruct((S, B, D), _F32),
                   jax.ShapeDtypeStruct((S, B, D), _F32)],
    )(gappt, gapst, pft, sft)


def _seq_body(combo_ref, xps_ref, pfg_ref, sfg_ref,
              pf1_ref, sf1_ref, af1_ref, ps1_ref, ss1_ref, as1_ref,
              afc_ref, p0_ref, s0_ref, a0_ref,
              olbas_ref, olbps_ref, olbss_ref,
              histp, hists, allst, gbp, gbs):
    t = pl.program_id(0)

    @pl.when(t == 0)
    def _init():
        histp[0] = jnp.broadcast_to(p0_ref[:], (B, D))
        hists[0] = jnp.broadcast_to(s0_ref[:], (B, D))
        allst[:] = jnp.broadcast_to(a0_ref[:], (B, D))

    def bbody(b, carry):
        v = combo_ref[b, t]
        gbp[b, :] = histp[v >> 8, b, :]
        gbs[b, :] = hists[v & 255, b, :]
        return carry

    lax.fori_loop(0, B, bbody, 0, unroll=True)
    gp = gbp[:]
    gs = gbs[:]

    pf = jax.nn.sigmoid(
        jnp.dot(gp, pf1_ref[:], preferred_element_type=_F32) + pfg_ref[0])
    lbps = gp * pf
    sf = jax.nn.sigmoid(
        jnp.dot(gs, sf1_ref[:], preferred_element_type=_F32) + sfg_ref[0])
    lbss = gs * sf
    a = allst[:]
    af = jax.nn.sigmoid(
        jnp.dot(a, af1_ref[:], preferred_element_type=_F32) + afc_ref[:])
    lbas = a * af
    olbas_ref[0] = lbas
    olbps_ref[0] = lbps
    olbss_ref[0] = lbss
    x = xps_ref[0]
    allst[:] = lbas + jnp.tanh(
        jnp.dot(lbas, as1_ref[:], preferred_element_type=_F32) + x[:, 0:D])
    newp = lbps + jnp.tanh(
        jnp.dot(lbps, ps1_ref[:], preferred_element_type=_F32) + x[:, D:2 * D])
    news = lbss + jnp.tanh(
        jnp.dot(lbss, ss1_ref[:], preferred_element_type=_F32) + x[:, 2 * D:])
    histp[t] = newp
    hists[t] = news


def _seq(combo, xps, pfg3, sfg3, pf1, sf1, af1, ps1, ss1, as1,
         afc, p0, s0, a0):
    row3 = pl.BlockSpec((1, B, D), lambda t, _c: (t, 0, 0))
    full = lambda shp: pl.BlockSpec(shp, lambda t, _c: tuple(0 for _ in shp))
    grid_spec = pltpu.PrefetchScalarGridSpec(
        num_scalar_prefetch=1,
        grid=(S,),
        in_specs=[pl.BlockSpec((1, B, 3 * D), lambda t, _c: (t, 0, 0)),
                  row3, row3,
                  full((D, D)), full((D, D)), full((D, D)), full((D, D)),
                  full((D, D)), full((D, D)), full((1, D)), full((1, D)),
                  full((1, D)), full((1, D))],
        out_specs=[row3, row3, row3],
        scratch_shapes=[
            pltpu.VMEM((S, B, D), _F32),
            pltpu.VMEM((S, B, D), _F32),
            pltpu.VMEM((B, D), _F32),
            pltpu.VMEM((B, D), _F32),
            pltpu.VMEM((B, D), _F32),
        ],
    )
    return pl.pallas_call(
        _seq_body,
        grid_spec=grid_spec,
        out_shape=[jax.ShapeDtypeStruct((S, B, D), _F32)] * 3,
    )(combo, xps, pfg3, sfg3, pf1, sf1, af1, ps1, ss1, as1, afc,
      p0, s0, a0)


def _head_body(lbas_ref, lbps_ref, lbss_ref, outx_ref, w1_ref, w2_ref,
               b2_ref, p_ref):
    m = jnp.concatenate([lbas_ref[:], lbps_ref[:], lbss_ref[:]],
                        axis=2).reshape(_TB * B, 3 * D)
    h = jnp.maximum(
        jnp.dot(m, w1_ref[:], preferred_element_type=_F32)
        + outx_ref[:].reshape(_TB * B, D), 0.0)
    p = jax.nn.sigmoid(
        jnp.dot(h, w2_ref[:], preferred_element_type=_F32) + b2_ref[:])
    p_ref[:] = p.reshape(_TB, B)


def _head(lbas, lbps, lbss, outx, w1abc, w2, b2):
    grid = (pl.cdiv(S, _TB),)
    row3 = pl.BlockSpec((_TB, B, D), lambda i: (i, 0, 0))
    full = lambda shp: pl.BlockSpec(shp, lambda i: tuple(0 for _ in shp))
    return pl.pallas_call(
        _head_body,
        grid=grid,
        in_specs=[row3, row3, row3, row3,
                  full((3 * D, D)), full((D, 1)), full((1, 1))],
        out_specs=pl.BlockSpec((_TB, B), lambda i: (i, 0)),
        out_shape=jax.ShapeDtypeStruct((S, B), _F32),
    )(lbas, lbps, lbss, outx, w1abc, w2, b2)


def _flatpad(x_bs):
    f = x_bs.T.reshape(N)
    f = jnp.concatenate([f, jnp.zeros((_NP - N,), jnp.int32)])
    return f.reshape(_IDXR, _CHUNK)


def kernel(qseqs, cseqs, rseqs, shft_qseqs, shft_cseqs, shft_rseqs,
           pro_embed, skill_embed, ans_embed, time_embed, ls_state,
           pro_state_init, skill_state_init, akt_pro_diff, akt_pro_change,
           out_w1, out_b1, out_w2, out_b2, pf_w, pf_b, ps_w, ps_b, af_w,
           af_b, sf_w, sf_b, ss_w, ss_b, as_w, as_b):
    pf1 = pf_w[:, :D].T
    sf1 = sf_w[:, :D].T
    af1 = af_w[:, :D].T
    ps1 = ps_w[:, :D].T
    ss1 = ss_w[:, :D].T
    as1 = as_w[:, :D].T
    pfw2 = pf_w[:, D:].T
    sfw2 = sf_w[:, D:].T
    afw2 = af_w[:, D:].T
    c3 = jnp.concatenate([as_w[:, D:].T, ps_w[:, D:].T, ss_w[:, D:].T], 1)
    b3 = jnp.concatenate([as_b, ps_b, ss_b]).reshape(1, 3 * D)
    w1abc = out_w1[:, :3 * D].T
    w1d = out_w1[:, 3 * D:].T
    b1 = out_b1.reshape(1, D)
    w2 = out_w2.T
    b2 = out_b2.reshape(1, 1)

    combo, gapp, gaps = _lastocc(shft_qseqs, shft_cseqs)

    pft, sft, afc, ansc3 = _tables(time_embed, pfw2, sfw2, afw2,
                                   pf_b.reshape(1, D), sf_b.reshape(1, D),
                                   af_b.reshape(1, D), ans_embed, c3)

    qf = _flatpad(shft_qseqs)
    qhif = _flatpad(shft_qseqs >> 7)
    cf = _flatpad(shft_cseqs)
    dif2d = jnp.concatenate(
        [akt_pro_diff.reshape(-1),
         jnp.zeros((782 * 128 - akt_pro_diff.shape[0],), _F32)]).reshape(
             782, 128)
    pro_r, skl_r, chg_r, difr_r = _sc_gather(
        qf, qhif, cf, pro_embed, skill_embed, akt_pro_change, dif2d)

    pfg3, sfg3 = _gaprows(gapp.T, gaps.T, pft, sft)

    rt = shft_rseqs.T
    qlane = (shft_qseqs & 127).T
    xps, outx = _xform(pro_r, skl_r, chg_r, difr_r, qlane, rt, c3, w1d, b3,
                       b1, ansc3)

    lbas, lbps, lbss = _seq(combo, xps, pfg3, sfg3, pf1, sf1,
                            af1, ps1, ss1, as1, afc, pro_state_init[0:1],
                            skill_state_init[0:1], ls_state)

    pt = _head(lbas, lbps, lbss, outx, w1abc, w2, b2)
    return pt.T


# speedup vs baseline: 1.0329x; 1.0104x over previous
import functools

import jax
import jax.numpy as jnp
from jax import lax
from jax.experimental import pallas as pl
from jax.experimental.pallas import tpu as pltpu
from jax.experimental.pallas import tpu_sc as plsc

B = 128
S = 199
D = 128
N = S * B

_NW = 32
_CHUNK = 64
_CPW = 14
_ROWS_W = _CHUNK * _CPW
_NP = _NW * _ROWS_W
_IDXR = _NP // _CHUNK

_F32 = jnp.float32


_BC = 8


def _lastocc_body(q_ref, c_ref, combo_ref, gapp_ref, gaps_ref):
    q = q_ref[:]
    c = c_ref[:]
    tpr = lax.broadcasted_iota(jnp.int32, (_BC, S, S), 2)
    tcu = lax.broadcasted_iota(jnp.int32, (_BC, S, S), 1)
    lt = tpr < tcu
    eqq = (q[:, :, None] == q[:, None, :]) & lt
    lbp = jnp.max(jnp.where(eqq, tpr, 0), axis=2)
    eqc = (c[:, :, None] == c[:, None, :]) & lt
    lbs = jnp.max(jnp.where(eqc, tpr, 0), axis=2)
    tv = lax.broadcasted_iota(jnp.int32, (_BC, S), 1)
    combo_ref[:] = lbp * 256 + lbs
    gapp_ref[:] = tv - lbp
    gaps_ref[:] = tv - lbs


def _lastocc(q, c):
    grid = (B // _BC,)
    spec = pl.BlockSpec((_BC, S), lambda i: (i, 0))
    return pl.pallas_call(
        _lastocc_body,
        grid=grid,
        in_specs=[spec, spec],
        out_specs=[spec, spec, spec],
        out_shape=[jax.ShapeDtypeStruct((B, S), jnp.int32)] * 3,
    )(q, c)


def _tables_body(te_ref, pfw2_ref, sfw2_ref, afw2_ref, pfb_ref, sfb_ref,
                 afb_ref, ans_ref, c3_ref, pft_ref, sft_ref, afc_ref,
                 ansc3_ref):
    te = te_ref[:]
    pft_ref[:] = jnp.dot(te, pfw2_ref[:], preferred_element_type=_F32) + pfb_ref[:]
    sft_ref[:] = jnp.dot(te, sfw2_ref[:], preferred_element_type=_F32) + sfb_ref[:]
    afc_ref[:] = jnp.dot(te[1:2], afw2_ref[:], preferred_element_type=_F32) + afb_ref[:]
    ansc3_ref[:] = jnp.dot(ans_ref[:], c3_ref[:], preferred_element_type=_F32)


def _tables(time_embed, pfw2, sfw2, afw2, pfb, sfb, afb, ans_embed, c3):
    return pl.pallas_call(
        _tables_body,
        out_shape=[
            jax.ShapeDtypeStruct((200, D), _F32),
            jax.ShapeDtypeStruct((200, D), _F32),
            jax.ShapeDtypeStruct((1, D), _F32),
            jax.ShapeDtypeStruct((2, 3 * D), _F32),
        ],
    )(time_embed, pfw2, sfw2, afw2, pfb, sfb, afb, ans_embed, c3)


def _sc_gather(qf, qhif, cf, pro_embed, skill_embed, akt_change,
               akt_diff2d):
    mesh = plsc.VectorSubcoreMesh(core_axis_name="c", subcore_axis_name="s")

    @functools.partial(
        pl.kernel,
        mesh=mesh,
        out_type=[
            jax.ShapeDtypeStruct((_NP, D), _F32),
            jax.ShapeDtypeStruct((_NP, D), _F32),
            jax.ShapeDtypeStruct((_NP, D), _F32),
            jax.ShapeDtypeStruct((_NP, D), _F32),
        ],
        scratch_types=[
            pltpu.VMEM((_CPW, _CHUNK), jnp.int32),
            pltpu.VMEM((_CPW, _CHUNK), jnp.int32),
            pltpu.VMEM((_CPW, _CHUNK), jnp.int32),
            pltpu.VMEM((3, _CHUNK, D), _F32),
            pltpu.VMEM((3, _CHUNK, D), _F32),
            pltpu.VMEM((3, _CHUNK, D), _F32),
            pltpu.VMEM((3, _CHUNK, D), _F32),
            pltpu.SemaphoreType.DMA,
            pltpu.SemaphoreType.DMA,
            pltpu.SemaphoreType.DMA,
            pltpu.SemaphoreType.DMA,
        ],
    )
    def k(qh, qhih, ch, proh, sklh, chgh, difh,
          opro, oskl, ochg, odif,
          qi, qhii, ci, bpro, bskl, bchg, bdif, gsem0, gsem1, gsem2, wsem):
        wid = lax.axis_index("s") * 2 + lax.axis_index("c")
        pltpu.sync_copy(qh.at[wid], qi)
        pltpu.sync_copy(qhih.at[wid], qhii)
        pltpu.sync_copy(ch.at[wid], ci)
        gsems = (gsem0, gsem1, gsem2)

        def fire(c, s):
            gs = gsems[s]
            return (pltpu.async_copy(proh.at[qi.at[c]], bpro.at[s], gs),
                    pltpu.async_copy(sklh.at[ci.at[c]], bskl.at[s], gs),
                    pltpu.async_copy(chgh.at[ci.at[c]], bchg.at[s], gs),
                    pltpu.async_copy(difh.at[qhii.at[c]], bdif.at[s], gs))

        nbuf = 3
        pend = {0: fire(0, 0), 1: fire(1, 1)}
        wrs = {0: [], 1: [], 2: []}
        for c in range(_CPW):
            s = c % nbuf
            for d in pend.pop(s):
                d.wait()
            if c + 2 < _CPW:
                sn = (c + 2) % nbuf
                for w in wrs[sn]:
                    w.wait()
                wrs[sn] = []
                pend[sn] = fire(c + 2, sn)
            dst = pl.ds(wid * _ROWS_W + c * _CHUNK, _CHUNK)
            wrs[s] = [
                pltpu.async_copy(bpro.at[s], opro.at[dst], wsem),
                pltpu.async_copy(bskl.at[s], oskl.at[dst], wsem),
                pltpu.async_copy(bchg.at[s], ochg.at[dst], wsem),
                pltpu.async_copy(bdif.at[s], odif.at[dst], wsem),
            ]
        for ws in wrs.values():
            for w in ws:
                w.wait()

    return k(qf, qhif, cf, pro_embed, skill_embed, akt_change, akt_diff2d)


_TB = 8


def _xform_body(pro_ref, skl_ref, chg_ref, difr_ref, qlane_ref, r_ref,
                c3_ref, w1d_ref, b3_ref, b1_ref, ansc3_ref, xps_ref,
                outx_ref):
    lane = lax.broadcasted_iota(jnp.int32, (_TB, B, D), 2)
    dmask = lane == qlane_ref[:][:, :, None]
    dif = jnp.sum(jnp.where(dmask, difr_ref[:].reshape(_TB, B, D), 0.0),
                  axis=2, keepdims=True)
    npe3 = (pro_ref[:].reshape(_TB, B, D) + skl_ref[:].reshape(_TB, B, D)
            + dif * chg_ref[:].reshape(_TB, B, D))
    npe2 = npe3.reshape(_TB * B, D)
    outx2 = jnp.dot(npe2, w1d_ref[:], preferred_element_type=_F32) + b1_ref[:]
    ansc = ansc3_ref[:]
    r = r_ref[:]
    sel = jnp.where(r[:, :, None] > 0, ansc[1:2][None], ansc[0:1][None])
    xyz2 = (jnp.dot(npe2, c3_ref[:], preferred_element_type=_F32)
            + b3_ref[:] + sel.reshape(_TB * B, 3 * D))
    xps_ref[:] = xyz2.reshape(_TB, B, 3 * D)
    outx_ref[:] = outx2.reshape(_TB, B, D)


def _xform(pro_r, skl_r, chg_r, difr_r, qlane, rt, c3, w1d, b3, b1, ansc3):
    grid = (pl.cdiv(S, _TB),)
    row3 = pl.BlockSpec((_TB, B, D), lambda i: (i, 0, 0))
    rowf = pl.BlockSpec((_TB * B, D), lambda i: (i, 0))
    row2 = pl.BlockSpec((_TB, B), lambda i: (i, 0))
    full = lambda shp: pl.BlockSpec(shp, lambda i: tuple(0 for _ in shp))
    return pl.pallas_call(
        _xform_body,
        grid=grid,
        in_specs=[rowf, rowf, rowf, rowf, row2, row2,
                  full((D, 3 * D)), full((D, D)), full((1, 3 * D)),
                  full((1, D)), full((2, 3 * D))],
        out_specs=[pl.BlockSpec((_TB, B, 3 * D), lambda i: (i, 0, 0)), row3],
        out_shape=[jax.ShapeDtypeStruct((S, B, 3 * D), _F32),
                   jax.ShapeDtypeStruct((S, B, D), _F32)],
    )(pro_r, skl_r, chg_r, difr_r, qlane, rt, c3, w1d, b3, b1, ansc3)


def _gaprows_body(gapp_ref, gaps_ref, pft_ref, sft_ref, pfg_ref, sfg_ref):
    giota = lax.broadcasted_iota(jnp.int32, (_TB, B, 200), 2)
    ohp = (gapp_ref[:][:, :, None] == giota).astype(_F32).reshape(
        _TB * B, 200)
    ohs = (gaps_ref[:][:, :, None] == giota).astype(_F32).reshape(
        _TB * B, 200)
    pfg_ref[:] = jnp.dot(ohp, pft_ref[:],
                         preferred_element_type=_F32).reshape(_TB, B, D)
    sfg_ref[:] = jnp.dot(ohs, sft_ref[:],
                         preferred_element_type=_F32).reshape(_TB, B, D)


def _gaprows(gappt, gapst, pft, sft):
    grid = (pl.cdiv(S, _TB),)
    row3 = pl.BlockSpec((_TB, B, D), lambda i: (i, 0, 0))
    row2 = pl.BlockSpec((_TB, B), lambda i: (i, 0))
    full = lambda shp: pl.BlockSpec(shp, lambda i: tuple(0 for _ in shp))
    return pl.pallas_call(
        _gaprows_body,
        grid=grid,
        in_specs=[row2, row2, full((200, D)), full((200, D))],
        out_specs=[row3, row3],
        out_shape=[jax.ShapeDtypeStruct((S, B, D), _F32),
                   jax.ShapeDtypeStruct((S, B, D), _F32)],
    )(gappt, gapst, pft, sft)


def _seq_body(combo_ref, xps_ref, pfg_ref, sfg_ref,
              pf1_ref, sf1_ref, af1_ref, ps1_ref, ss1_ref, as1_ref,
              afc_ref, p0_ref, s0_ref, a0_ref,
              olbas_ref, olbps_ref, olbss_ref,
              histp, hists, allst, gbp, gbs):
    t = pl.program_id(0)

    @pl.when(t == 0)
    def _init():
        histp[0] = jnp.broadcast_to(p0_ref[:], (B, D))
        hists[0] = jnp.broadcast_to(s0_ref[:], (B, D))
        allst[:] = jnp.broadcast_to(a0_ref[:], (B, D))

    def bbody(b, carry):
        v = combo_ref[b, t]
        gbp[b, :] = histp[v >> 8, b, :]
        gbs[b, :] = hists[v & 255, b, :]
        return carry

    lax.fori_loop(0, B, bbody, 0, unroll=True)
    gp = gbp[:]
    gs = gbs[:]

    pf = jax.nn.sigmoid(
        jnp.dot(gp, pf1_ref[:], preferred_element_type=_F32) + pfg_ref[0])
    lbps = gp * pf
    sf = jax.nn.sigmoid(
        jnp.dot(gs, sf1_ref[:], preferred_element_type=_F32) + sfg_ref[0])
    lbss = gs * sf
    a = allst[:]
    af = jax.nn.sigmoid(
        jnp.dot(a, af1_ref[:], preferred_element_type=_F32) + afc_ref[:])
    lbas = a * af
    olbas_ref[0] = lbas
    olbps_ref[0] = lbps
    olbss_ref[0] = lbss
    x = xps_ref[0]
    allst[:] = lbas + jnp.tanh(
        jnp.dot(lbas, as1_ref[:], preferred_element_type=_F32) + x[:, 0:D])
    newp = lbps + jnp.tanh(
        jnp.dot(lbps, ps1_ref[:], preferred_element_type=_F32) + x[:, D:2 * D])
    news = lbss + jnp.tanh(
        jnp.dot(lbss, ss1_ref[:], preferred_element_type=_F32) + x[:, 2 * D:])
    histp[t] = newp
    hists[t] = news


def _seq(combo, xps, pfg3, sfg3, pf1, sf1, af1, ps1, ss1, as1,
         afc, p0, s0, a0):
    row3 = pl.BlockSpec((1, B, D), lambda t, _c: (t, 0, 0))
    full = lambda shp: pl.BlockSpec(shp, lambda t, _c: tuple(0 for _ in shp))
    grid_spec = pltpu.PrefetchScalarGridSpec(
        num_scalar_prefetch=1,
        grid=(S,),
        in_specs=[pl.BlockSpec((1, B, 3 * D), lambda t, _c: (t, 0, 0)),
                  row3, row3,
                  full((D, D)), full((D, D)), full((D, D)), full((D, D)),
                  full((D, D)), full((D, D)), full((1, D)), full((1, D)),
                  full((1, D)), full((1, D))],
        out_specs=[row3, row3, row3],
        scratch_shapes=[
            pltpu.VMEM((S, B, D), _F32),
            pltpu.VMEM((S, B, D), _F32),
            pltpu.VMEM((B, D), _F32),
            pltpu.VMEM((B, D), _F32),
            pltpu.VMEM((B, D), _F32),
        ],
    )
    return pl.pallas_call(
        _seq_body,
        grid_spec=grid_spec,
        out_shape=[jax.ShapeDtypeStruct((S, B, D), _F32)] * 3,
    )(combo, xps, pfg3, sfg3, pf1, sf1, af1, ps1, ss1, as1, afc,
      p0, s0, a0)


def _head_body(lbas_ref, lbps_ref, lbss_ref, outx_ref, w1_ref, w2_ref,
               b2_ref, p_ref):
    m = jnp.concatenate([lbas_ref[:], lbps_ref[:], lbss_ref[:]],
                        axis=2).reshape(_TB * B, 3 * D)
    h = jnp.maximum(
        jnp.dot(m, w1_ref[:], preferred_element_type=_F32)
        + outx_ref[:].reshape(_TB * B, D), 0.0)
    p = jax.nn.sigmoid(
        jnp.dot(h, w2_ref[:], preferred_element_type=_F32) + b2_ref[:])
    p_ref[:] = p.reshape(_TB, B)


def _head(lbas, lbps, lbss, outx, w1abc, w2, b2):
    grid = (pl.cdiv(S, _TB),)
    row3 = pl.BlockSpec((_TB, B, D), lambda i: (i, 0, 0))
    full = lambda shp: pl.BlockSpec(shp, lambda i: tuple(0 for _ in shp))
    return pl.pallas_call(
        _head_body,
        grid=grid,
        in_specs=[row3, row3, row3, row3,
                  full((3 * D, D)), full((D, 1)), full((1, 1))],
        out_specs=pl.BlockSpec((_TB, B), lambda i: (i, 0)),
        out_shape=jax.ShapeDtypeStruct((S, B), _F32),
    )(lbas, lbps, lbss, outx, w1abc, w2, b2)


def _flatpad(x_bs):
    f = x_bs.T.reshape(N)
    f = jnp.concatenate([f, jnp.zeros((_NP - N,), jnp.int32)])
    return f.reshape(_NW, _CPW, _CHUNK)


def kernel(qseqs, cseqs, rseqs, shft_qseqs, shft_cseqs, shft_rseqs,
           pro_embed, skill_embed, ans_embed, time_embed, ls_state,
           pro_state_init, skill_state_init, akt_pro_diff, akt_pro_change,
           out_w1, out_b1, out_w2, out_b2, pf_w, pf_b, ps_w, ps_b, af_w,
           af_b, sf_w, sf_b, ss_w, ss_b, as_w, as_b):
    pf1 = pf_w[:, :D].T
    sf1 = sf_w[:, :D].T
    af1 = af_w[:, :D].T
    ps1 = ps_w[:, :D].T
    ss1 = ss_w[:, :D].T
    as1 = as_w[:, :D].T
    pfw2 = pf_w[:, D:].T
    sfw2 = sf_w[:, D:].T
    afw2 = af_w[:, D:].T
    c3 = jnp.concatenate([as_w[:, D:].T, ps_w[:, D:].T, ss_w[:, D:].T], 1)
    b3 = jnp.concatenate([as_b, ps_b, ss_b]).reshape(1, 3 * D)
    w1abc = out_w1[:, :3 * D].T
    w1d = out_w1[:, 3 * D:].T
    b1 = out_b1.reshape(1, D)
    w2 = out_w2.T
    b2 = out_b2.reshape(1, 1)

    combo, gapp, gaps = _lastocc(shft_qseqs, shft_cseqs)

    pft, sft, afc, ansc3 = _tables(time_embed, pfw2, sfw2, afw2,
                                   pf_b.reshape(1, D), sf_b.reshape(1, D),
                                   af_b.reshape(1, D), ans_embed, c3)

    qf = _flatpad(shft_qseqs)
    qhif = _flatpad(shft_qseqs >> 7)
    cf = _flatpad(shft_cseqs)
    dif2d = jnp.concatenate(
        [akt_pro_diff.reshape(-1),
         jnp.zeros((782 * 128 - akt_pro_diff.shape[0],), _F32)]).reshape(
             782, 128)
    pro_r, skl_r, chg_r, difr_r = _sc_gather(
        qf, qhif, cf, pro_embed, skill_embed, akt_pro_change, dif2d)

    pfg3, sfg3 = _gaprows(gapp.T, gaps.T, pft, sft)

    rt = shft_rseqs.T
    qlane = (shft_qseqs & 127).T
    xps, outx = _xform(pro_r, skl_r, chg_r, difr_r, qlane, rt, c3, w1d, b3,
                       b1, ansc3)

    lbas, lbps, lbss = _seq(combo, xps, pfg3, sfg3, pf1, sf1,
                            af1, ps1, ss1, as1, afc, pro_state_init[0:1],
                            skill_state_init[0:1], ls_state)

    pt = _head(lbas, lbps, lbss, outx, w1abc, w2, b2)
    return pt.T
